# baseline (speedup 1.0000x reference)
import jax
import jax.numpy as jnp
from jax import lax
from jax.experimental import pallas as pl
from jax.experimental.pallas import tpu as pltpu
from jax.experimental.pallas import tpu_sc as plsc

EMBED_DIM = 16
PACK = 128 // EMBED_DIM
NUM_WORKERS = 32
WINDOW = 512
NBUF = 4


def kernel(x, table):
    batch, seq = x.shape
    num_indices = batch * seq
    num_rows = table.shape[0]
    indices = x.T.reshape((num_indices,)).astype(jnp.int32)

    rows_per_worker = num_indices // NUM_WORKERS
    chunks = rows_per_worker // WINDOW
    outer = chunks // NBUF

    mesh = plsc.VectorSubcoreMesh(core_axis_name="c", subcore_axis_name="s")

    @pl.kernel(
        out_type=jax.ShapeDtypeStruct((seq, 2, batch // 128, 8, 128),
                                      table.dtype),
        mesh=mesh,
        compiler_params=pltpu.CompilerParams(use_tc_tiling_on_sc=False,
                                             needs_layout_passes=False),
        scratch_types=[
            pltpu.VMEM((NBUF, WINDOW), jnp.int32),
            pltpu.VMEM((NBUF, WINDOW, EMBED_DIM), jnp.float32),
            pltpu.VMEM((NBUF, EMBED_DIM, WINDOW + 1), jnp.float32),
            pltpu.SemaphoreType.DMA((NBUF,)),
            pltpu.SemaphoreType.DMA((NBUF,)),
            pltpu.SemaphoreType.DMA((NBUF,)),
        ],
    )
    def gather_kernel(table_hbm, i_hbm, o_hbm, idx_v, rows_v, trans_v,
                      sem_i, sem_g, sem_o):
        wid = lax.axis_index("s") * 2 + lax.axis_index("c")
        wbase = wid * rows_per_worker

        def start_idx(b, c):
            pltpu.async_copy(i_hbm.at[pl.ds(wbase + c * WINDOW, WINDOW)],
                             idx_v.at[b], sem_i.at[b])

        def wait_idx(b):
            pltpu.make_async_copy(i_hbm.at[pl.ds(0, WINDOW)],
                                  idx_v.at[b], sem_i.at[b]).wait()

        def start_gather(b):
            pltpu.async_copy(table_hbm.at[idx_v.at[b]], rows_v.at[b], sem_g.at[b])

        def wait_gather(b):
            pltpu.make_async_copy(table_hbm.at[idx_v.at[b]],
                                  rows_v.at[b], sem_g.at[b]).wait()

        def start_out(b, c):
            flat = wbase + c * WINDOW
            s = flat // batch
            blk0 = (flat % batch) // 128
            for a in range(2):
                for bb in range(WINDOW // 128):
                    pltpu.async_copy(
                        trans_v.at[b, pl.ds(8 * a, 8), pl.ds(128 * bb, 128)],
                        o_hbm.at[s, a, blk0 + bb],
                        sem_o.at[b])

        def wait_out(b):
            for _ in range(2 * (WINDOW // 128)):
                pltpu.make_async_copy(
                    trans_v.at[b, pl.ds(0, 8), pl.ds(0, 128)],
                    o_hbm.at[0, 0, 0],
                    sem_o.at[b]).wait()

        def transpose(b):
            f_ids = lax.iota(jnp.int32, 16)
            zeros = jnp.full((16,), 0, jnp.int32)

            @plsc.parallel_loop(0, WINDOW, 16, unroll=4)
            def _(j0):
                col_base = zeros + j0
                for dj in range(16):
                    vec = rows_v[b, j0 + dj, :]
                    plsc.store_scatter(trans_v.at[b], [f_ids, col_base + dj], vec)

        for b in range(NBUF):
            start_idx(b, b)
        for b in range(NBUF):
            wait_idx(b)
            start_gather(b)

        @pl.loop(0, outer - 1)
        def _(o):
            cbase = o * NBUF
            for b in range(NBUF):
                wait_gather(b)
                transpose(b)
                start_out(b, cbase + b)
                start_idx(b, cbase + NBUF + b)
            for b in range(NBUF):
                wait_out(b)
                wait_idx(b)
                start_gather(b)

        for b in range(NBUF):
            wait_gather(b)
            transpose(b)
            start_out(b, (outer - 1) * NBUF + b)
        for b in range(NBUF):
            wait_out(b)

    out = gather_kernel(table, indices)
    return out.transpose(2, 4, 0, 1, 3).reshape(batch, seq, EMBED_DIM)


# speedup vs baseline: 1.1316x; 1.1316x over previous
import jax
import jax.numpy as jnp
from jax import lax
from jax.experimental import pallas as pl
from jax.experimental.pallas import tpu as pltpu
from jax.experimental.pallas import tpu_sc as plsc

EMBED_DIM = 16
PACK = 128 // EMBED_DIM
NUM_WORKERS = 32
WINDOW = 512
NBUF = 4


def kernel(x, table):
    batch, seq = x.shape
    num_indices = batch * seq
    num_rows = table.shape[0]
    indices = x.T.reshape((num_indices,)).astype(jnp.int32)

    rows_per_worker = num_indices // NUM_WORKERS
    chunks = rows_per_worker // WINDOW
    outer = chunks // NBUF

    mesh = plsc.VectorSubcoreMesh(core_axis_name="c", subcore_axis_name="s")

    @pl.kernel(
        out_type=jax.ShapeDtypeStruct((seq, 2, batch // 128, 8, 128),
                                      table.dtype),
        mesh=mesh,
        compiler_params=pltpu.CompilerParams(use_tc_tiling_on_sc=False,
                                             needs_layout_passes=False),
        scratch_types=[
            pltpu.VMEM((NBUF, WINDOW), jnp.int32),
            pltpu.VMEM((NBUF, WINDOW, EMBED_DIM), jnp.float32),
            pltpu.VMEM((NBUF, EMBED_DIM, WINDOW + 1), jnp.float32),
            pltpu.SemaphoreType.DMA((NBUF,)),
            pltpu.SemaphoreType.DMA((NBUF,)),
            pltpu.SemaphoreType.DMA((NBUF,)),
        ],
    )
    def gather_kernel(table_hbm, i_hbm, o_hbm, idx_v, rows_v, trans_v,
                      sem_i, sem_g, sem_o):
        wid = lax.axis_index("s") * 2 + lax.axis_index("c")
        wbase = wid * rows_per_worker

        def start_idx(b, c):
            pltpu.async_copy(i_hbm.at[pl.ds(wbase + c * WINDOW, WINDOW)],
                             idx_v.at[b], sem_i.at[b])

        def wait_idx(b):
            pltpu.make_async_copy(i_hbm.at[pl.ds(0, WINDOW)],
                                  idx_v.at[b], sem_i.at[b]).wait()

        def start_gather(b):
            pltpu.async_copy(table_hbm.at[idx_v.at[b]], rows_v.at[b], sem_g.at[b])

        def wait_gather(b):
            pltpu.make_async_copy(table_hbm.at[idx_v.at[b]],
                                  rows_v.at[b], sem_g.at[b]).wait()

        def start_out(b, c):
            flat = wbase + c * WINDOW
            s = flat // batch
            blk0 = (flat % batch) // 128
            for a in range(2):
                for bb in range(WINDOW // 128):
                    pltpu.async_copy(
                        trans_v.at[b, pl.ds(8 * a, 8), pl.ds(128 * bb, 128)],
                        o_hbm.at[s, a, blk0 + bb],
                        sem_o.at[b])

        def wait_out(b):
            for _ in range(2 * (WINDOW // 128)):
                pltpu.make_async_copy(
                    trans_v.at[b, pl.ds(0, 8), pl.ds(0, 128)],
                    o_hbm.at[0, 0, 0],
                    sem_o.at[b]).wait()

        def transpose(b):
            f_ids = lax.iota(jnp.int32, 16)
            zeros = jnp.full((16,), 0, jnp.int32)

            @plsc.parallel_loop(0, WINDOW, 16, unroll=2)
            def _(j0):
                col_base = zeros + j0
                for dj in range(16):
                    vec = rows_v[b, j0 + dj, :]
                    plsc.store_scatter(trans_v.at[b], [f_ids, col_base + dj], vec)

        for b in range(NBUF):
            start_idx(b, b)
        for b in range(NBUF):
            wait_idx(b)
            start_gather(b)

        @pl.loop(0, outer - 1)
        def _(o):
            cbase = o * NBUF
            for b in range(NBUF):
                wait_gather(b)
                transpose(b)
                start_out(b, cbase + b)
                start_idx(b, cbase + NBUF + b)
            for b in range(NBUF):
                wait_out(b)
                wait_idx(b)
                start_gather(b)

        for b in range(NBUF):
            wait_gather(b)
            transpose(b)
            start_out(b, (outer - 1) * NBUF + b)
        for b in range(NBUF):
            wait_out(b)

    out = gather_kernel(table, indices)
    return out.transpose(2, 4, 0, 1, 3).reshape(batch, seq, EMBED_DIM)
